# baseline (speedup 1.0000x reference)
import functools

import jax
import jax.numpy as jnp
from jax import lax
from jax.experimental import pallas as pl
from jax.experimental.pallas import tpu as pltpu
from jax.experimental.pallas import tpu_sc as plsc

E = 8
H = 1024
NC = 2
NS = 16
NW = NC * NS
L = 16


def _logits_tc(hs2d, weight, nblk, blk):
    def body(w_ref, hs_ref, out_ref):
        out_ref[...] = lax.dot_general(
            w_ref[...], hs_ref[...],
            (((1,), (1,)), ((), ())),
            preferred_element_type=jnp.float32,
        )[None]

    return pl.pallas_call(
        body,
        grid=(nblk,),
        in_specs=[
            pl.BlockSpec((E, H), lambda i: (0, 0)),
            pl.BlockSpec((blk, H), lambda i: (i, 0)),
        ],
        out_specs=pl.BlockSpec((1, E, blk), lambda i: (i, 0, 0)),
        out_shape=jax.ShapeDtypeStruct((nblk, E, blk), jnp.float32),
    )(weight, hs2d)


def _route_sc(logits, n_tokens, chunk):
    mesh = plsc.VectorSubcoreMesh(core_axis_name="c", subcore_axis_name="s")

    @functools.partial(
        pl.kernel,
        mesh=mesh,
        out_type=[
            jax.ShapeDtypeStruct((2, n_tokens), jnp.int32),
            jax.ShapeDtypeStruct((2, n_tokens), jnp.float32),
        ],
        scratch_types=[
            pltpu.VMEM((E, chunk), jnp.float32),
            pltpu.VMEM((chunk,), jnp.int32),
            pltpu.VMEM((chunk,), jnp.int32),
            pltpu.VMEM((chunk,), jnp.float32),
            pltpu.VMEM((chunk,), jnp.float32),
        ],
    )
    def k(lg_hbm, idx_hbm, wgt_hbm, lg_v, i1_v, i2_v, w1_v, w2_v):
        wid = lax.axis_index("s") * NC + lax.axis_index("c")
        pltpu.sync_copy(lg_hbm.at[wid], lg_v)

        def body(i, carry):
            sl = pl.ds(i * L, L)
            ls = [lg_v[e, sl] for e in range(E)]
            m1 = ls[0]
            a1 = jnp.zeros((L,), jnp.int32)
            for e in range(1, E):
                gt = ls[e] > m1
                m1 = jnp.where(gt, ls[e], m1)
                a1 = jnp.where(gt, e, a1)
            m2 = jnp.full((L,), -jnp.inf, jnp.float32)
            a2 = jnp.zeros((L,), jnp.int32)
            for e in range(E):
                gt = jnp.logical_and(a1 != e, ls[e] > m2)
                m2 = jnp.where(gt, ls[e], m2)
                a2 = jnp.where(gt, e, a2)
            r = jnp.exp(m2 - m1)
            w1 = 1.0 / (1.0 + r)
            i1_v[sl] = a1
            i2_v[sl] = a2
            w1_v[sl] = w1
            w2_v[sl] = 1.0 - w1
            return carry

        lax.fori_loop(0, chunk // L, body, 0)
        base = wid * chunk
        pltpu.sync_copy(i1_v, idx_hbm.at[0, pl.ds(base, chunk)])
        pltpu.sync_copy(i2_v, idx_hbm.at[1, pl.ds(base, chunk)])
        pltpu.sync_copy(w1_v, wgt_hbm.at[0, pl.ds(base, chunk)])
        pltpu.sync_copy(w2_v, wgt_hbm.at[1, pl.ds(base, chunk)])

    return k(logits)


def kernel(hidden_states, weight):
    bsz, seq_len, h = hidden_states.shape
    n = bsz * seq_len
    hs2d = hidden_states.reshape(n, h)
    chunk = n // NW
    logits = _logits_tc(hs2d, weight, NW, chunk)
    idx2n, wgt2n = _route_sc(logits, n, chunk)
    return idx2n.T, wgt2n.T


# speedup vs baseline: 1.1209x; 1.1209x over previous
import functools

import jax
import jax.numpy as jnp
from jax import lax
from jax.experimental import pallas as pl
from jax.experimental.pallas import tpu as pltpu
from jax.experimental.pallas import tpu_sc as plsc

E = 8
H = 1024
NC = 2
NS = 16
NW = NC * NS
L = 16


def _logits_tc(hs2d, weight, blk):
    n = hs2d.shape[0]

    def body(w_ref, hs_ref, out_ref):
        out_ref[...] = lax.dot_general(
            w_ref[...], hs_ref[...],
            (((1,), (1,)), ((), ())),
            preferred_element_type=jnp.float32,
        )

    return pl.pallas_call(
        body,
        grid=(n // blk,),
        in_specs=[
            pl.BlockSpec((E, H), lambda i: (0, 0)),
            pl.BlockSpec((blk, H), lambda i: (i, 0)),
        ],
        out_specs=pl.BlockSpec((E, blk), lambda i: (0, i)),
        out_shape=jax.ShapeDtypeStruct((E, n), jnp.float32),
    )(weight, hs2d)


def _route_sc(logits, n_tokens, chunk):
    mesh = plsc.VectorSubcoreMesh(core_axis_name="c", subcore_axis_name="s")

    @functools.partial(
        pl.kernel,
        mesh=mesh,
        out_type=[
            jax.ShapeDtypeStruct((2, n_tokens), jnp.int32),
            jax.ShapeDtypeStruct((2, n_tokens), jnp.float32),
        ],
        scratch_types=[
            pltpu.VMEM((E, chunk), jnp.float32),
            pltpu.VMEM((chunk,), jnp.int32),
            pltpu.VMEM((chunk,), jnp.int32),
            pltpu.VMEM((chunk,), jnp.float32),
            pltpu.VMEM((chunk,), jnp.float32),
        ],
    )
    def k(lg_hbm, idx_hbm, wgt_hbm, lg_v, i1_v, i2_v, w1_v, w2_v):
        wid = lax.axis_index("s") * NC + lax.axis_index("c")
        pltpu.sync_copy(lg_hbm.at[:, pl.ds(wid * chunk, chunk)], lg_v)

        def body(i, carry):
            sl = pl.ds(i * L, L)
            ls = [lg_v[e, sl] for e in range(E)]
            m1 = ls[0]
            a1 = jnp.zeros((L,), jnp.int32)
            for e in range(1, E):
                gt = ls[e] > m1
                m1 = jnp.where(gt, ls[e], m1)
                a1 = jnp.where(gt, e, a1)
            m2 = jnp.full((L,), -jnp.inf, jnp.float32)
            a2 = jnp.zeros((L,), jnp.int32)
            for e in range(E):
                gt = jnp.logical_and(a1 != e, ls[e] > m2)
                m2 = jnp.where(gt, ls[e], m2)
                a2 = jnp.where(gt, e, a2)
            r = jnp.exp(m2 - m1)
            w1 = 1.0 / (1.0 + r)
            i1_v[sl] = a1
            i2_v[sl] = a2
            w1_v[sl] = w1
            w2_v[sl] = 1.0 - w1
            return carry

        lax.fori_loop(0, chunk // L, body, 0)
        base = wid * chunk
        pltpu.sync_copy(i1_v, idx_hbm.at[0, pl.ds(base, chunk)])
        pltpu.sync_copy(i2_v, idx_hbm.at[1, pl.ds(base, chunk)])
        pltpu.sync_copy(w1_v, wgt_hbm.at[0, pl.ds(base, chunk)])
        pltpu.sync_copy(w2_v, wgt_hbm.at[1, pl.ds(base, chunk)])

    return k(logits)


TC_BLK = 2048


def kernel(hidden_states, weight):
    bsz, seq_len, h = hidden_states.shape
    n = bsz * seq_len
    hs2d = hidden_states.reshape(n, h)
    chunk = n // NW
    logits = _logits_tc(hs2d, weight, TC_BLK)
    idx2n, wgt2n = _route_sc(logits, n, chunk)
    return idx2n.T, wgt2n.T
